# baseline (device time: 22135 ns/iter reference)
import jax
import jax.numpy as jnp
from jax import lax
from jax.experimental import pallas as pl
from jax.experimental.pallas import tpu as pltpu

N_DEV = 4


def kernel(A, B):
    m_per, k = A.shape
    _, n = B.shape

    def body(a_ref, b_ref, out_ref, comm_ref, send_sems, recv_sems):
        my_pos = lax.axis_index("i")
        left = (my_pos - 1) % N_DEV
        right = (my_pos + 1) % N_DEV

        barrier_sem = pltpu.get_barrier_semaphore()
        for nbr in [left, right]:
            pl.semaphore_signal(
                barrier_sem, inc=1,
                device_id=(nbr,), device_id_type=pl.DeviceIdType.MESH,
            )
        pl.semaphore_wait(barrier_sem, 2)

        a_bf = a_ref[...].astype(jnp.bfloat16)
        b_bf = b_ref[...].astype(jnp.bfloat16)
        comm_ref[0] = a_bf
        out_ref[pl.ds(my_pos * m_per, m_per), :] = jnp.dot(
            a_bf, b_bf, preferred_element_type=jnp.float32
        )

        for h in range(N_DEV - 1):
            rdma = pltpu.make_async_remote_copy(
                src_ref=comm_ref.at[h],
                dst_ref=comm_ref.at[h + 1],
                send_sem=send_sems.at[h],
                recv_sem=recv_sems.at[h + 1],
                device_id=(right,),
                device_id_type=pl.DeviceIdType.MESH,
            )
            rdma.start()
            rdma.wait()

            origin = (my_pos - h - 1) % N_DEV
            out_ref[pl.ds(origin * m_per, m_per), :] = jnp.dot(
                comm_ref[h + 1], b_bf, preferred_element_type=jnp.float32
            )

    return pl.pallas_call(
        body,
        out_shape=jax.ShapeDtypeStruct((N_DEV * m_per, n), jnp.float32),
        in_specs=[
            pl.BlockSpec(memory_space=pltpu.VMEM),
            pl.BlockSpec(memory_space=pltpu.VMEM),
        ],
        out_specs=pl.BlockSpec(memory_space=pltpu.VMEM),
        scratch_shapes=[
            pltpu.VMEM((N_DEV, m_per, k), jnp.bfloat16),
            pltpu.SemaphoreType.DMA((N_DEV,)),
            pltpu.SemaphoreType.DMA((N_DEV,)),
        ],
        compiler_params=pltpu.CompilerParams(collective_id=0),
    )(A, B)


# device time: 15035 ns/iter; 1.4722x vs baseline; 1.4722x over previous
import jax
import jax.numpy as jnp
from jax import lax
from jax.experimental import pallas as pl
from jax.experimental.pallas import tpu as pltpu

N_DEV = 4


def kernel(A, B):
    m_per, k = A.shape
    _, n = B.shape

    def body(a_ref, b_ref, out_ref, own_ref, comm_ref, send_sems, recv_sems):
        my_pos = lax.axis_index("i")
        left = (my_pos - 1) % N_DEV
        right = (my_pos + 1) % N_DEV
        diag = (my_pos + 2) % N_DEV

        barrier_sem = pltpu.get_barrier_semaphore()
        for nbr in [left, right, diag]:
            pl.semaphore_signal(
                barrier_sem, inc=1,
                device_id=(nbr,), device_id_type=pl.DeviceIdType.MESH,
            )
        pl.semaphore_wait(barrier_sem, 3)

        own_ref[...] = a_ref[...].astype(jnp.bfloat16)

        rdmas = []
        for slot, tgt in ((0, right), (1, left), (2, diag)):
            r = pltpu.make_async_remote_copy(
                src_ref=own_ref,
                dst_ref=comm_ref.at[slot],
                send_sem=send_sems.at[slot],
                recv_sem=recv_sems.at[slot],
                device_id=(tgt,),
                device_id_type=pl.DeviceIdType.MESH,
            )
            r.start()
            rdmas.append(r)

        b_bf = b_ref[...].astype(jnp.bfloat16)
        out_ref[pl.ds(my_pos * m_per, m_per), :] = jnp.dot(
            own_ref[...], b_bf, preferred_element_type=jnp.float32
        )

        for slot, origin in ((0, left), (1, right), (2, diag)):
            rdmas[slot].wait_recv()
            out_ref[pl.ds(origin * m_per, m_per), :] = jnp.dot(
                comm_ref[slot], b_bf, preferred_element_type=jnp.float32
            )

        for r in rdmas:
            r.wait_send()

    return pl.pallas_call(
        body,
        out_shape=jax.ShapeDtypeStruct((N_DEV * m_per, n), jnp.float32),
        in_specs=[
            pl.BlockSpec(memory_space=pltpu.VMEM),
            pl.BlockSpec(memory_space=pltpu.VMEM),
        ],
        out_specs=pl.BlockSpec(memory_space=pltpu.VMEM),
        scratch_shapes=[
            pltpu.VMEM((m_per, k), jnp.bfloat16),
            pltpu.VMEM((3, m_per, k), jnp.bfloat16),
            pltpu.SemaphoreType.DMA((3,)),
            pltpu.SemaphoreType.DMA((3,)),
        ],
        compiler_params=pltpu.CompilerParams(collective_id=0),
    )(A, B)
